# baseline (speedup 1.0000x reference)
import functools
import jax
import jax.numpy as jnp
from jax import lax
from jax.experimental import pallas as pl
from jax.experimental.pallas import tpu as pltpu
from jax.experimental.pallas import tpu_sc as plsc

_SIG_WEIGHT = 30.0
_CLOSE_MIN = 0.05

_W = 224
_H = 224
_NW = 16
_CPR = _W // 16
_RPW = 4
_TC_ROW0 = _NW * _RPW


def _lane_shift(x, idx):
    return lax.gather(
        x, idx[:, None],
        dimension_numbers=lax.GatherDimensionNumbers(
            offset_dims=(), collapsed_slice_dims=(0,), start_index_map=(0,)),
        slice_sizes=(1,),
        mode=lax.GatherScatterMode.PROMISE_IN_BOUNDS)


def _sc_body(r_hbm, a_hbm, out_hbm, a_b, r_b, acc_v, sem):
    wid = lax.axis_index("s")
    rows = pl.ds((wid // 2) * 8, 8)
    copies = [
        pltpu.async_copy(a_hbm.at[0, 0, rows, pl.ds(0, 128)],
                         a_b.at[:, pl.ds(0, 128)], sem),
        pltpu.async_copy(a_hbm.at[0, 0, rows, pl.ds(128, 96)],
                         a_b.at[:, pl.ds(128, 96)], sem),
        pltpu.async_copy(r_hbm.at[0, 0, rows, pl.ds(0, 128)],
                         r_b.at[:, pl.ds(0, 128)], sem),
        pltpu.async_copy(r_hbm.at[0, 0, rows, pl.ds(128, 96)],
                         r_b.at[:, pl.ds(128, 96)], sem),
    ]
    for cp in copies:
        cp.wait()

    zero = jnp.zeros((16,), jnp.float32)
    r0 = lax.rem(wid, 2) * _RPW

    def chunk_loss(col, r, a, an, ap):
        vn_m = (col < _W - 1) & (an != 0.0)
        vs = a != 0.0
        vs_m = (col >= 1) & vs
        vp_m = (col >= 2) & (ap != 0.0)
        m = jnp.where(vn_m, an, jnp.where(vs_m, a, jnp.where(vp_m, ap, a)))
        d0 = r - a
        orig_mse = d0 * d0
        dm = r - m
        alt = dm * dm * dm + _CLOSE_MIN
        loss = jnp.minimum(orig_mse, alt)
        return jnp.where(vs, loss * _SIG_WEIGHT, loss)

    def do_row(ri, accs):
        i = r0 + ri
        lane = lax.iota(jnp.int32, 16)
        idx_sl = jnp.minimum(lane + 1, 15)
        idx_sr = jnp.maximum(lane - 1, 0)

        def one_chunk(ch, acc, a, hi):
            o16 = 16 * ch
            a_next = a_b[i, pl.ds(o16 + 16, 16)]
            r = r_b[i, pl.ds(o16, 16)]
            col = lane + o16
            sl = _lane_shift(a, idx_sl)
            lo = _lane_shift(a_next, lane * 0)
            an = jnp.where(lane < 15, sl, lo)
            ap = jnp.where(lane > 0, _lane_shift(a, idx_sr), hi)
            loss = chunk_loss(col, r, a, an, ap)
            hi_next = _lane_shift(a, lane * 0 + 15)
            return acc + loss, a_next, hi_next

        def step(k, carry):
            acc0, acc1, a, hi = carry
            acc0, a, hi = one_chunk(2 * k, acc0, a, hi)
            acc1, a, hi = one_chunk(2 * k + 1, acc1, a, hi)
            return acc0, acc1, a, hi

        a0 = a_b[i, pl.ds(0, 16)]
        acc0, acc1, a12, hi12 = lax.fori_loop(
            0, (_CPR - 2) // 2, step, (accs[0], accs[1], a0, zero))
        acc0, a13, hi13 = one_chunk(_CPR - 2, acc0, a12, hi12)
        r13 = r_b[i, pl.ds(16 * (_CPR - 1), 16)]
        col = lane + 16 * (_CPR - 1)
        an = _lane_shift(a13, idx_sl)
        ap = jnp.where(lane > 0, _lane_shift(a13, idx_sr), hi13)
        loss = chunk_loss(col, r13, a13, an, ap)
        return acc0 + loss, acc1

    acc0, acc1 = lax.fori_loop(0, _RPW, do_row, (zero, zero))
    acc_v[...] = acc0 + acc1
    pltpu.sync_copy(acc_v, out_hbm.at[wid])


def _tc_kernel(r_ref, a_ref, o_ref):
    img = pl.program_id(0)
    a = a_ref[0, 0]
    r = r_ref[0, 0]
    row = jax.lax.broadcasted_iota(jnp.int32, a.shape, 0)
    col = jax.lax.broadcasted_iota(jnp.int32, a.shape, 1)
    a_next = jnp.concatenate([a[:, 1:], a[:, :1]], axis=1)
    a_prev = jnp.concatenate([a[:, -1:], a[:, :-1]], axis=1)
    valid_next = (col < _W - 1) & (a_next != 0.0)
    valid_self = (col >= 1) & (a != 0.0)
    valid_prev = (col >= 2) & (a_prev != 0.0)
    m = jnp.where(valid_next, a_next,
                  jnp.where(valid_self, a,
                            jnp.where(valid_prev, a_prev, a)))
    d0 = r - a
    orig_mse = d0 * d0
    dm = r - m
    alt = dm * dm * dm + _CLOSE_MIN
    loss = jnp.minimum(orig_mse, alt)
    loss = jnp.where(a != 0.0, loss * _SIG_WEIGHT, loss)
    row0 = jnp.where(img == 0, _TC_ROW0, 0)
    loss = jnp.where(row >= row0, loss, 0.0)
    s = jnp.sum(loss)

    @pl.when(img == 0)
    def _():
        o_ref[0, 0] = s

    @pl.when(img == 1)
    def _():
        o_ref[0, 0] = o_ref[0, 0] + s


def kernel(reconstruction, original):
    mesh = plsc.VectorSubcoreMesh(core_axis_name="c", subcore_axis_name="s",
                                  num_cores=1)
    sc_fn = functools.partial(
        pl.kernel, mesh=mesh,
        out_type=jax.ShapeDtypeStruct((_NW, 16), jnp.float32),
        scratch_types=[
            pltpu.VMEM((8, _W), jnp.float32),
            pltpu.VMEM((8, _W), jnp.float32),
            pltpu.VMEM((16,), jnp.float32),
            pltpu.SemaphoreType.DMA,
        ],
    )(_sc_body)
    partials = sc_fn(reconstruction, original)

    img = pl.BlockSpec((1, 1, _H, _W), lambda i: (i, 0, 0, 0))
    tc_sum = pl.pallas_call(
        _tc_kernel,
        grid=(2,),
        out_shape=jax.ShapeDtypeStruct((1, 1), jnp.float32),
        in_specs=[img, img],
        out_specs=pl.BlockSpec((1, 1), lambda i: (0, 0),
                               memory_space=pltpu.SMEM),
    )(reconstruction, original)

    return (jnp.sum(partials) + tc_sum[0, 0]) / (2 * _H * _W)
